# baseline (speedup 1.0000x reference)
import dataclasses
import functools

import jax
import jax.numpy as jnp
from jax import lax
from jax.experimental import pallas as pl
from jax.experimental.pallas import tpu as pltpu
from jax.experimental.pallas import tpu_sc as plsc

_PX = 512
_PY = 512
_P = _PX * _PY
_B = 2
_N = 100000
_C = 64
_W = 128

_NC = 2
_NS = 16
_LANES = 16

_NPAD = 100352
_SHARE = _NPAD // _NS
_NVREG = _SHARE // _LANES

_CH = 8192
_CHBITS = 13
_HALF = _P // _NC
_NCHUNK = _HALF // _CH

_K = 128
_KBITS = 7
_BINROWS = (_SHARE + _NCHUNK * (_K - 1)) // _K + 1
_ZROWS = _CH // _NS

_SENTINEL = 1 << 30


def _sc_body(x_hbm, idx_hbm, zrow_hbm, out_hbm,
             idx_v, cid_v, lidx_v, rows_a, rows_b, cnt_v, ssem, chunk_sh):
    core = lax.axis_index("c")
    sub = lax.axis_index("s")
    iota = lax.broadcasted_iota(jnp.int32, (_LANES,), 0)
    rowbase = sub * _ZROWS
    zero16 = jnp.zeros((_LANES,), jnp.int32)
    cbase = core * _HALF

    pltpu.sync_copy(zrow_hbm, chunk_sh.at[pl.ds(rowbase, _ZROWS)])
    plsc.subcore_barrier()

    for b in range(_B):
        pltpu.sync_copy(idx_hbm.at[b, pl.ds(sub * _SHARE, _SHARE)], idx_v)

        _hist = jax.named_scope("bin_hist")
        _hist.__enter__()
        cnt_v[pl.ds(0, _LANES)] = zero16
        cnt_v[pl.ds(_LANES, _LANES)] = zero16
        cnt_v[pl.ds(2 * _LANES, _LANES)] = zero16

        def hbody(i, _):
            iv = idx_v[pl.ds(i * _LANES, _LANES)]
            m = jnp.logical_and(iv >= cbase, iv < cbase + _HALF)
            bk = jnp.where(m, (iv - cbase) >> _CHBITS, 0)
            occ, lastm = plsc.scan_count(bk, m)
            plsc.addupdate_scatter(cnt_v, [bk], occ + 1,
                                   mask=jnp.logical_and(lastm, m))
            return 0

        lax.fori_loop(0, _NVREG, hbody, 0)
        _hist.__exit__(None, None, None)

        c0 = cnt_v[pl.ds(0, _LANES)]
        c1 = cnt_v[pl.ds(_LANES, _LANES)]
        a0 = ((c0 + (_K - 1)) >> _KBITS) << _KBITS
        a1 = ((c1 + (_K - 1)) >> _KBITS) << _KBITS
        e0 = plsc.cumsum(a0) - a0
        t0 = jnp.sum(a0)
        e1 = plsc.cumsum(a1) - a1 + t0
        cnt_v[pl.ds(0, _LANES)] = e0
        cnt_v[pl.ds(_LANES, _LANES)] = e1

        _pre = jax.named_scope("bin_prefill")
        _pre.__enter__()
        dummy_lidx = _CH + sub * _LANES + iota

        def fbody(i, _):
            r = i >> 3
            c = (i & 7) * _LANES
            rv = jnp.full((_LANES,), r, jnp.int32)
            cv = c + iota
            plsc.store_scatter(lidx_v, [rv, cv], dummy_lidx)
            plsc.store_scatter(cid_v, [rv, cv], iota)
            return 0

        lax.fori_loop(0, _BINROWS * 8, fbody, 0)
        _pre.__exit__(None, None, None)

        _sb = jax.named_scope("bin_scatter")
        _sb.__enter__()

        def sbody(i, _):
            iv = idx_v[pl.ds(i * _LANES, _LANES)]
            m = jnp.logical_and(iv >= cbase, iv < cbase + _HALF)
            l = iv - cbase
            bk = jnp.where(m, l >> _CHBITS, 0)
            occ, lastm = plsc.scan_count(bk, m)
            basev = plsc.load_gather(cnt_v, [bk], mask=m)
            pos = basev + occ
            pid = sub * _SHARE + i * _LANES + iota
            plsc.store_scatter(cid_v, [pos >> 7, pos & 127], pid, mask=m)
            plsc.store_scatter(lidx_v, [pos >> 7, pos & 127],
                               l & (_CH - 1), mask=m)
            plsc.addupdate_scatter(cnt_v, [bk], occ + 1,
                                   mask=jnp.logical_and(lastm, m))
            return 0

        lax.fori_loop(0, _NVREG, sbody, 0)
        _sb.__exit__(None, None, None)

        _cp = jax.named_scope("chunk_passes")
        _cp.__enter__()

        @pl.loop(0, _NCHUNK)
        def _chunk_pass(cc):
            base = cbase + cc * _CH
            lane = cc & (_LANES - 1)
            hi = cc >= _LANES
            ev = jnp.where(hi, e1, e0)
            av = jnp.where(hi, a1, a0)
            onlane = iota == lane
            start = jnp.sum(jnp.where(onlane, ev, 0))
            srow = start >> 7
            nrows = jnp.sum(jnp.where(onlane, av, 0)) >> 7

            def gpair(jj, _):
                r0 = srow + jj * 2
                r1 = r0 + 1

                @pl.when(jj > 0)
                def _w0():
                    pltpu.make_async_copy(
                        rows_a, chunk_sh.at[lidx_v.at[r0 - 2]], ssem).wait()

                pltpu.sync_copy(x_hbm.at[b].at[cid_v.at[r0]], rows_a)
                pltpu.async_copy(rows_a, chunk_sh.at[lidx_v.at[r0]], ssem,
                                 add=True)

                @pl.when(jj * 2 + 1 < nrows)
                def _hb():
                    @pl.when(jj > 0)
                    def _w1():
                        pltpu.make_async_copy(
                            rows_b, chunk_sh.at[lidx_v.at[r1 - 2]], ssem).wait()

                    pltpu.sync_copy(x_hbm.at[b].at[cid_v.at[r1]], rows_b)
                    pltpu.async_copy(rows_b, chunk_sh.at[lidx_v.at[r1]], ssem,
                                     add=True)

                return 0

            _s1 = jax.named_scope("seg_streams")
            _s1.__enter__()
            lax.fori_loop(0, (nrows + 1) >> 1, gpair, 0)

            @pl.when(nrows >= 1)
            def _ta():
                pltpu.make_async_copy(
                    rows_a, chunk_sh.at[lidx_v.at[srow]], ssem).wait()

            @pl.when(nrows >= 2)
            def _tb():
                pltpu.make_async_copy(
                    rows_b, chunk_sh.at[lidx_v.at[srow]], ssem).wait()

            _s1.__exit__(None, None, None)

            _s2 = jax.named_scope("bar1")
            _s2.__enter__()
            plsc.subcore_barrier()
            _s2.__exit__(None, None, None)
            _s3 = jax.named_scope("copyout")
            _s3.__enter__()
            pltpu.sync_copy(chunk_sh.at[pl.ds(rowbase, _ZROWS)],
                            out_hbm.at[b].at[pl.ds(base + rowbase, _ZROWS)])
            _s3.__exit__(None, None, None)
            _s4 = jax.named_scope("zero")
            _s4.__enter__()
            pltpu.sync_copy(zrow_hbm, chunk_sh.at[pl.ds(rowbase, _ZROWS)])
            _s4.__exit__(None, None, None)
            _s5 = jax.named_scope("bar2")
            _s5.__enter__()
            plsc.subcore_barrier()
            _s5.__exit__(None, None, None)

        _cp.__exit__(None, None, None)


_T_BLK = 8192
_PAD_BLK = 10000


def _transpose_body(in_ref, out_ref):
    out_ref[0, :, :] = in_ref[0, :, 0:_C].T


def _pad_body(x_ref, out_ref):
    out_ref[0, :, 0:_C] = x_ref[0]
    out_ref[0, :, _C:_W] = jnp.zeros((_PAD_BLK, _W - _C), jnp.float32)


def kernel(x, indices):
    x_pad = pl.pallas_call(
        _pad_body,
        grid=(_B, _N // _PAD_BLK),
        in_specs=[pl.BlockSpec((1, _PAD_BLK, _C), lambda b, n: (b, n, 0))],
        out_specs=pl.BlockSpec((1, _PAD_BLK, _W), lambda b, n: (b, n, 0)),
        out_shape=jax.ShapeDtypeStruct((_B, _N, _W), jnp.float32),
    )(x)
    idx_pad = jnp.pad(indices, ((0, 0), (0, _NPAD - _N)),
                      constant_values=_SENTINEL)
    zrow = jnp.zeros((_ZROWS, _W), jnp.float32)

    cp = pltpu.CompilerParams()
    if "needs_layout_passes" in pltpu.CompilerParams.__dataclass_fields__:
        cp = dataclasses.replace(cp, needs_layout_passes=False)
    mesh = plsc.VectorSubcoreMesh(core_axis_name="c", subcore_axis_name="s")
    grid_pw = pl.kernel(
        _sc_body,
        out_type=jax.ShapeDtypeStruct((_B, _P, _W), jnp.float32),
        mesh=mesh,
        scratch_types=[
            pltpu.VMEM((_SHARE,), jnp.int32),
            pltpu.VMEM((_BINROWS, _K), jnp.int32),
            pltpu.VMEM((_BINROWS, _K), jnp.int32),
            pltpu.VMEM((_K, _W), jnp.float32),
            pltpu.VMEM((_K, _W), jnp.float32),
            pltpu.VMEM((3 * _LANES,), jnp.int32),
            pltpu.SemaphoreType.DMA,
            pltpu.VMEM_SHARED((_CH + _NS * 16, _W), jnp.float32),
        ],
        compiler_params=cp,
    )(x_pad, idx_pad, zrow)

    out = pl.pallas_call(
        _transpose_body,
        grid=(_B, _P // _T_BLK),
        in_specs=[pl.BlockSpec((1, _T_BLK, _W), lambda b, t: (b, t, 0))],
        out_specs=pl.BlockSpec((1, _C, _T_BLK), lambda b, t: (b, 0, t)),
        out_shape=jax.ShapeDtypeStruct((_B, _C, _P), jnp.float32),
    )(grid_pw)
    return out.reshape(_B, _C, _PX, _PY)


# speedup vs baseline: 1.0016x; 1.0016x over previous
import dataclasses
import functools

import jax
import jax.numpy as jnp
from jax import lax
from jax.experimental import pallas as pl
from jax.experimental.pallas import tpu as pltpu
from jax.experimental.pallas import tpu_sc as plsc

_PX = 512
_PY = 512
_P = _PX * _PY
_B = 2
_N = 100000
_C = 64
_W = 128

_NC = 2
_NS = 16
_LANES = 16

_NPAD = 100352
_SHARE = _NPAD // _NS
_NVREG = _SHARE // _LANES

_CH = 8192
_CHBITS = 13
_HALF = _P // _NC
_NCHUNK = _HALF // _CH

_K = 128
_KBITS = 7
_BINROWS = (_SHARE + _NCHUNK * (_K - 1)) // _K + 1
_ZROWS = _CH // _NS

_SENTINEL = 1 << 30


def _sc_body(x_hbm, idx_hbm, zrow_hbm, out_hbm,
             idx_v, cid_v, lidx_v, rows_a, rows_b, cnt_v, ssem, gsem,
             chunk_sh):
    core = lax.axis_index("c")
    sub = lax.axis_index("s")
    iota = lax.broadcasted_iota(jnp.int32, (_LANES,), 0)
    rowbase = sub * _ZROWS
    zero16 = jnp.zeros((_LANES,), jnp.int32)
    cbase = core * _HALF

    pltpu.sync_copy(zrow_hbm, chunk_sh.at[pl.ds(rowbase, _ZROWS)])
    plsc.subcore_barrier()

    for b in range(_B):
        pltpu.sync_copy(idx_hbm.at[b, pl.ds(sub * _SHARE, _SHARE)], idx_v)

        _hist = jax.named_scope("bin_hist")
        _hist.__enter__()
        cnt_v[pl.ds(0, _LANES)] = zero16
        cnt_v[pl.ds(_LANES, _LANES)] = zero16
        cnt_v[pl.ds(2 * _LANES, _LANES)] = zero16

        def hbody(i, _):
            iv = idx_v[pl.ds(i * _LANES, _LANES)]
            m = jnp.logical_and(iv >= cbase, iv < cbase + _HALF)
            bk = jnp.where(m, (iv - cbase) >> _CHBITS, 0)
            occ, lastm = plsc.scan_count(bk, m)
            plsc.addupdate_scatter(cnt_v, [bk], occ + 1,
                                   mask=jnp.logical_and(lastm, m))
            return 0

        lax.fori_loop(0, _NVREG, hbody, 0)
        _hist.__exit__(None, None, None)

        c0 = cnt_v[pl.ds(0, _LANES)]
        c1 = cnt_v[pl.ds(_LANES, _LANES)]
        a0 = ((c0 + (_K - 1)) >> _KBITS) << _KBITS
        a1 = ((c1 + (_K - 1)) >> _KBITS) << _KBITS
        e0 = plsc.cumsum(a0) - a0
        t0 = jnp.sum(a0)
        e1 = plsc.cumsum(a1) - a1 + t0
        cnt_v[pl.ds(0, _LANES)] = e0
        cnt_v[pl.ds(_LANES, _LANES)] = e1

        _pre = jax.named_scope("bin_prefill")
        _pre.__enter__()
        dummy_lidx = _CH + sub * _LANES + iota

        def fbody(i, _):
            r = i >> 3
            c = (i & 7) * _LANES
            rv = jnp.full((_LANES,), r, jnp.int32)
            cv = c + iota
            plsc.store_scatter(lidx_v, [rv, cv], dummy_lidx)
            plsc.store_scatter(cid_v, [rv, cv], iota)
            return 0

        lax.fori_loop(0, _BINROWS * 8, fbody, 0)
        _pre.__exit__(None, None, None)

        _sb = jax.named_scope("bin_scatter")
        _sb.__enter__()

        def sbody(i, _):
            iv = idx_v[pl.ds(i * _LANES, _LANES)]
            m = jnp.logical_and(iv >= cbase, iv < cbase + _HALF)
            l = iv - cbase
            bk = jnp.where(m, l >> _CHBITS, 0)
            occ, lastm = plsc.scan_count(bk, m)
            basev = plsc.load_gather(cnt_v, [bk], mask=m)
            pos = basev + occ
            pid = sub * _SHARE + i * _LANES + iota
            plsc.store_scatter(cid_v, [pos >> 7, pos & 127], pid, mask=m)
            plsc.store_scatter(lidx_v, [pos >> 7, pos & 127],
                               l & (_CH - 1), mask=m)
            plsc.addupdate_scatter(cnt_v, [bk], occ + 1,
                                   mask=jnp.logical_and(lastm, m))
            return 0

        lax.fori_loop(0, _NVREG, sbody, 0)
        _sb.__exit__(None, None, None)

        _cp = jax.named_scope("chunk_passes")
        _cp.__enter__()

        @pl.loop(0, _NCHUNK)
        def _chunk_pass(cc):
            base = cbase + cc * _CH
            lane = cc & (_LANES - 1)
            hi = cc >= _LANES
            ev = jnp.where(hi, e1, e0)
            av = jnp.where(hi, a1, a0)
            onlane = iota == lane
            start = jnp.sum(jnp.where(onlane, ev, 0))
            srow = start >> 7
            nrows = jnp.sum(jnp.where(onlane, av, 0)) >> 7

            def gpair(jj, _):
                r0 = srow + jj * 2
                r1 = r0 + 1
                has_b = jj * 2 + 1 < nrows

                @pl.when(jj > 0)
                def _w0():
                    pltpu.make_async_copy(
                        rows_a, chunk_sh.at[lidx_v.at[r0 - 2]], ssem).wait()

                pltpu.async_copy(x_hbm.at[b].at[cid_v.at[r0]], rows_a, gsem)

                @pl.when(has_b)
                def _hb0():
                    @pl.when(jj > 0)
                    def _w1():
                        pltpu.make_async_copy(
                            rows_b, chunk_sh.at[lidx_v.at[r1 - 2]], ssem).wait()

                    pltpu.async_copy(x_hbm.at[b].at[cid_v.at[r1]], rows_b,
                                     gsem)

                pltpu.make_async_copy(
                    x_hbm.at[b].at[cid_v.at[r0]], rows_a, gsem).wait()
                pltpu.async_copy(rows_a, chunk_sh.at[lidx_v.at[r0]], ssem,
                                 add=True)

                @pl.when(has_b)
                def _hb1():
                    pltpu.make_async_copy(
                        x_hbm.at[b].at[cid_v.at[r1]], rows_b, gsem).wait()
                    pltpu.async_copy(rows_b, chunk_sh.at[lidx_v.at[r1]], ssem,
                                     add=True)

                return 0

            _s1 = jax.named_scope("seg_streams")
            _s1.__enter__()
            lax.fori_loop(0, (nrows + 1) >> 1, gpair, 0)

            @pl.when(nrows >= 1)
            def _ta():
                pltpu.make_async_copy(
                    rows_a, chunk_sh.at[lidx_v.at[srow]], ssem).wait()

            @pl.when(nrows >= 2)
            def _tb():
                pltpu.make_async_copy(
                    rows_b, chunk_sh.at[lidx_v.at[srow]], ssem).wait()

            _s1.__exit__(None, None, None)

            _s2 = jax.named_scope("bar1")
            _s2.__enter__()
            plsc.subcore_barrier()
            _s2.__exit__(None, None, None)
            _s3 = jax.named_scope("copyout")
            _s3.__enter__()
            pltpu.sync_copy(chunk_sh.at[pl.ds(rowbase, _ZROWS)],
                            out_hbm.at[b].at[pl.ds(base + rowbase, _ZROWS)])
            _s3.__exit__(None, None, None)
            _s4 = jax.named_scope("zero")
            _s4.__enter__()
            pltpu.sync_copy(zrow_hbm, chunk_sh.at[pl.ds(rowbase, _ZROWS)])
            _s4.__exit__(None, None, None)
            _s5 = jax.named_scope("bar2")
            _s5.__enter__()
            plsc.subcore_barrier()
            _s5.__exit__(None, None, None)

        _cp.__exit__(None, None, None)


_T_BLK = 8192
_PAD_BLK = 10000


def _transpose_body(in_ref, out_ref):
    out_ref[0, :, :] = in_ref[0, :, 0:_C].T


def _pad_body(x_ref, out_ref):
    out_ref[0, :, 0:_C] = x_ref[0]
    out_ref[0, :, _C:_W] = jnp.zeros((_PAD_BLK, _W - _C), jnp.float32)


def kernel(x, indices):
    x_pad = pl.pallas_call(
        _pad_body,
        grid=(_B, _N // _PAD_BLK),
        in_specs=[pl.BlockSpec((1, _PAD_BLK, _C), lambda b, n: (b, n, 0))],
        out_specs=pl.BlockSpec((1, _PAD_BLK, _W), lambda b, n: (b, n, 0)),
        out_shape=jax.ShapeDtypeStruct((_B, _N, _W), jnp.float32),
    )(x)
    idx_pad = jnp.pad(indices, ((0, 0), (0, _NPAD - _N)),
                      constant_values=_SENTINEL)
    zrow = jnp.zeros((_ZROWS, _W), jnp.float32)

    cp = pltpu.CompilerParams()
    if "needs_layout_passes" in pltpu.CompilerParams.__dataclass_fields__:
        cp = dataclasses.replace(cp, needs_layout_passes=False)
    mesh = plsc.VectorSubcoreMesh(core_axis_name="c", subcore_axis_name="s")
    grid_pw = pl.kernel(
        _sc_body,
        out_type=jax.ShapeDtypeStruct((_B, _P, _W), jnp.float32),
        mesh=mesh,
        scratch_types=[
            pltpu.VMEM((_SHARE,), jnp.int32),
            pltpu.VMEM((_BINROWS, _K), jnp.int32),
            pltpu.VMEM((_BINROWS, _K), jnp.int32),
            pltpu.VMEM((_K, _W), jnp.float32),
            pltpu.VMEM((_K, _W), jnp.float32),
            pltpu.VMEM((3 * _LANES,), jnp.int32),
            pltpu.SemaphoreType.DMA,
            pltpu.SemaphoreType.DMA,
            pltpu.VMEM_SHARED((_CH + _NS * 16, _W), jnp.float32),
        ],
        compiler_params=cp,
    )(x_pad, idx_pad, zrow)

    out = pl.pallas_call(
        _transpose_body,
        grid=(_B, _P // _T_BLK),
        in_specs=[pl.BlockSpec((1, _T_BLK, _W), lambda b, t: (b, t, 0))],
        out_specs=pl.BlockSpec((1, _C, _T_BLK), lambda b, t: (b, 0, t)),
        out_shape=jax.ShapeDtypeStruct((_B, _C, _P), jnp.float32),
    )(grid_pw)
    return out.reshape(_B, _C, _PX, _PY)


# speedup vs baseline: 1.4690x; 1.4667x over previous
import dataclasses
import functools

import jax
import jax.numpy as jnp
from jax import lax
from jax.experimental import pallas as pl
from jax.experimental.pallas import tpu as pltpu
from jax.experimental.pallas import tpu_sc as plsc

_PX = 512
_PY = 512
_P = _PX * _PY
_B = 2
_N = 100000
_C = 64
_W = 128

_NC = 2
_NS = 16
_LANES = 16

_NPAD = 100352
_SHARE = _NPAD // _NS
_NVREG = _SHARE // _LANES

_CH = 8192
_CHBITS = 13
_HALF = _P // _NC
_NCHUNK = _HALF // _CH

_K = 128
_KBITS = 7
_BINROWS = (_SHARE + _NCHUNK * (_K - 1)) // _K + 1
_ZROWS = _CH // _NS

_SENTINEL = 1 << 30


def _sc_body(x_hbm, idx_hbm, zrow_hbm, out_hbm,
             idx_v, cid_v, lidx_v, rows_a, rows_b, cnt_v, ssem, gsem,
             chunk_sh):
    core = lax.axis_index("c")
    sub = lax.axis_index("s")
    iota = lax.broadcasted_iota(jnp.int32, (_LANES,), 0)
    rowbase = sub * _ZROWS
    zero16 = jnp.zeros((_LANES,), jnp.int32)
    cbase = core * _HALF

    pltpu.sync_copy(zrow_hbm, chunk_sh.at[pl.ds(rowbase, _ZROWS)])
    plsc.subcore_barrier()

    for b in range(_B):
        pltpu.sync_copy(idx_hbm.at[b, pl.ds(sub * _SHARE, _SHARE)], idx_v)

        _hist = jax.named_scope("bin_hist")
        _hist.__enter__()
        cnt_v[pl.ds(0, _LANES)] = zero16
        cnt_v[pl.ds(_LANES, _LANES)] = zero16
        cnt_v[pl.ds(2 * _LANES, _LANES)] = zero16

        def hbody(i, _):
            iv = idx_v[pl.ds(i * _LANES, _LANES)]
            m = jnp.logical_and(iv >= cbase, iv < cbase + _HALF)
            bk = jnp.where(m, (iv - cbase) >> _CHBITS, 0)
            occ, lastm = plsc.scan_count(bk, m)
            plsc.addupdate_scatter(cnt_v, [bk], occ + 1,
                                   mask=jnp.logical_and(lastm, m))
            return 0

        lax.fori_loop(0, _NVREG, hbody, 0)
        _hist.__exit__(None, None, None)

        c0 = cnt_v[pl.ds(0, _LANES)]
        c1 = cnt_v[pl.ds(_LANES, _LANES)]
        a0 = ((c0 + (_K - 1)) >> _KBITS) << _KBITS
        a1 = ((c1 + (_K - 1)) >> _KBITS) << _KBITS
        e0 = plsc.cumsum(a0) - a0
        t0 = jnp.sum(a0)
        e1 = plsc.cumsum(a1) - a1 + t0
        cnt_v[pl.ds(0, _LANES)] = e0
        cnt_v[pl.ds(_LANES, _LANES)] = e1

        _pre = jax.named_scope("bin_prefill")
        _pre.__enter__()
        dummy_lidx = _CH + sub * _LANES + iota

        def fbody(i, _):
            r = i >> 3
            c = (i & 7) * _LANES
            rv = jnp.full((_LANES,), r, jnp.int32)
            cv = c + iota
            plsc.store_scatter(lidx_v, [rv, cv], dummy_lidx)
            plsc.store_scatter(cid_v, [rv, cv],
                               (r * _K + cv + sub * 509) & 8191)
            return 0

        lax.fori_loop(0, _BINROWS * 8, fbody, 0)
        _pre.__exit__(None, None, None)

        _sb = jax.named_scope("bin_scatter")
        _sb.__enter__()

        def sbody(i, _):
            iv = idx_v[pl.ds(i * _LANES, _LANES)]
            m = jnp.logical_and(iv >= cbase, iv < cbase + _HALF)
            l = iv - cbase
            bk = jnp.where(m, l >> _CHBITS, 0)
            occ, lastm = plsc.scan_count(bk, m)
            basev = plsc.load_gather(cnt_v, [bk], mask=m)
            pos = basev + occ
            pid = sub * _SHARE + i * _LANES + iota
            plsc.store_scatter(cid_v, [pos >> 7, pos & 127], pid, mask=m)
            plsc.store_scatter(lidx_v, [pos >> 7, pos & 127],
                               l & (_CH - 1), mask=m)
            plsc.addupdate_scatter(cnt_v, [bk], occ + 1,
                                   mask=jnp.logical_and(lastm, m))
            return 0

        lax.fori_loop(0, _NVREG, sbody, 0)
        _sb.__exit__(None, None, None)

        _cp = jax.named_scope("chunk_passes")
        _cp.__enter__()

        @pl.loop(0, _NCHUNK)
        def _chunk_pass(cc):
            base = cbase + cc * _CH
            lane = cc & (_LANES - 1)
            hi = cc >= _LANES
            ev = jnp.where(hi, e1, e0)
            av = jnp.where(hi, a1, a0)
            onlane = iota == lane
            start = jnp.sum(jnp.where(onlane, ev, 0))
            srow = start >> 7
            nrows = jnp.sum(jnp.where(onlane, av, 0)) >> 7

            def gpair(jj, _):
                r0 = srow + jj * 2
                r1 = r0 + 1
                has_b = jj * 2 + 1 < nrows

                @pl.when(jj > 0)
                def _w0():
                    pltpu.make_async_copy(
                        rows_a, chunk_sh.at[lidx_v.at[r0 - 2]], ssem).wait()

                pltpu.async_copy(x_hbm.at[b].at[cid_v.at[r0]], rows_a, gsem)

                @pl.when(has_b)
                def _hb0():
                    @pl.when(jj > 0)
                    def _w1():
                        pltpu.make_async_copy(
                            rows_b, chunk_sh.at[lidx_v.at[r1 - 2]], ssem).wait()

                    pltpu.async_copy(x_hbm.at[b].at[cid_v.at[r1]], rows_b,
                                     gsem)

                pltpu.make_async_copy(
                    x_hbm.at[b].at[cid_v.at[r0]], rows_a, gsem).wait()
                pltpu.async_copy(rows_a, chunk_sh.at[lidx_v.at[r0]], ssem,
                                 add=True)

                @pl.when(has_b)
                def _hb1():
                    pltpu.make_async_copy(
                        x_hbm.at[b].at[cid_v.at[r1]], rows_b, gsem).wait()
                    pltpu.async_copy(rows_b, chunk_sh.at[lidx_v.at[r1]], ssem,
                                     add=True)

                return 0

            _s1 = jax.named_scope("seg_streams")
            _s1.__enter__()
            lax.fori_loop(0, (nrows + 1) >> 1, gpair, 0)

            @pl.when(nrows >= 1)
            def _ta():
                pltpu.make_async_copy(
                    rows_a, chunk_sh.at[lidx_v.at[srow]], ssem).wait()

            @pl.when(nrows >= 2)
            def _tb():
                pltpu.make_async_copy(
                    rows_b, chunk_sh.at[lidx_v.at[srow]], ssem).wait()

            _s1.__exit__(None, None, None)

            _s2 = jax.named_scope("bar1")
            _s2.__enter__()
            plsc.subcore_barrier()
            _s2.__exit__(None, None, None)
            _s3 = jax.named_scope("copyout")
            _s3.__enter__()
            pltpu.sync_copy(chunk_sh.at[pl.ds(rowbase, _ZROWS)],
                            out_hbm.at[b].at[pl.ds(base + rowbase, _ZROWS)])
            _s3.__exit__(None, None, None)
            _s4 = jax.named_scope("zero")
            _s4.__enter__()
            pltpu.sync_copy(zrow_hbm, chunk_sh.at[pl.ds(rowbase, _ZROWS)])
            _s4.__exit__(None, None, None)
            _s5 = jax.named_scope("bar2")
            _s5.__enter__()
            plsc.subcore_barrier()
            _s5.__exit__(None, None, None)

        _cp.__exit__(None, None, None)


_T_BLK = 8192
_PAD_BLK = 10000


def _transpose_body(in_ref, out_ref):
    out_ref[0, :, :] = in_ref[0, :, 0:_C].T


def _pad_body(x_ref, out_ref):
    out_ref[0, :, 0:_C] = x_ref[0]
    out_ref[0, :, _C:_W] = jnp.zeros((_PAD_BLK, _W - _C), jnp.float32)


def kernel(x, indices):
    x_pad = pl.pallas_call(
        _pad_body,
        grid=(_B, _N // _PAD_BLK),
        in_specs=[pl.BlockSpec((1, _PAD_BLK, _C), lambda b, n: (b, n, 0))],
        out_specs=pl.BlockSpec((1, _PAD_BLK, _W), lambda b, n: (b, n, 0)),
        out_shape=jax.ShapeDtypeStruct((_B, _N, _W), jnp.float32),
    )(x)
    idx_pad = jnp.pad(indices, ((0, 0), (0, _NPAD - _N)),
                      constant_values=_SENTINEL)
    zrow = jnp.zeros((_ZROWS, _W), jnp.float32)

    cp = pltpu.CompilerParams()
    if "needs_layout_passes" in pltpu.CompilerParams.__dataclass_fields__:
        cp = dataclasses.replace(cp, needs_layout_passes=False)
    mesh = plsc.VectorSubcoreMesh(core_axis_name="c", subcore_axis_name="s")
    grid_pw = pl.kernel(
        _sc_body,
        out_type=jax.ShapeDtypeStruct((_B, _P, _W), jnp.float32),
        mesh=mesh,
        scratch_types=[
            pltpu.VMEM((_SHARE,), jnp.int32),
            pltpu.VMEM((_BINROWS, _K), jnp.int32),
            pltpu.VMEM((_BINROWS, _K), jnp.int32),
            pltpu.VMEM((_K, _W), jnp.float32),
            pltpu.VMEM((_K, _W), jnp.float32),
            pltpu.VMEM((3 * _LANES,), jnp.int32),
            pltpu.SemaphoreType.DMA,
            pltpu.SemaphoreType.DMA,
            pltpu.VMEM_SHARED((_CH + _NS * 16, _W), jnp.float32),
        ],
        compiler_params=cp,
    )(x_pad, idx_pad, zrow)

    out = pl.pallas_call(
        _transpose_body,
        grid=(_B, _P // _T_BLK),
        in_specs=[pl.BlockSpec((1, _T_BLK, _W), lambda b, t: (b, t, 0))],
        out_specs=pl.BlockSpec((1, _C, _T_BLK), lambda b, t: (b, 0, t)),
        out_shape=jax.ShapeDtypeStruct((_B, _C, _P), jnp.float32),
    )(grid_pw)
    return out.reshape(_B, _C, _PX, _PY)
